# Initial kernel scaffold; baseline (speedup 1.0000x reference)
import jax
import jax.numpy as jnp
from jax.experimental import pallas as pl


def kernel(node_inp, node_type, edge_index, edge_type, edge_sign, Wq, bq, Wk, bk, Wv, bv, rel_q, rel_k, rel_v, sign_k_fixed, sign_v_fixed, sign_k_neutral, sign_v_neutral, rel_bias, skip, gamma, beta):
    raise NotImplementedError("write your pallas kernel here")


# speedup vs baseline: 5.2006x; 5.2006x over previous
import functools
import math

import jax
import jax.numpy as jnp
from jax import lax
from jax.experimental import pallas as pl
from jax.experimental.pallas import tpu as pltpu
from jax.experimental.pallas import tpu_sc as plsc

N = 10000
E = 320000
DIM = 128
T = 4
R = 8
H = 8
DK = 16

NC = 2
NS = 16
NW = NC * NS
C = 64
NCHUNK = -(-E // (NW * C))
EPW = NCHUNK * C
EP = NW * EPW
ACC_W = 144
NPAD = N + 16
ROWS_PT = N // NS

_mesh = plsc.VectorSubcoreMesh(core_axis_name="c", subcore_axis_name="s")


def _proj_body(x_ref, t_ref, wq_ref, bq_ref, wk_ref, bk_ref, wv_ref, bv_ref,
               q_ref, k_ref, v_ref):
    x = x_ref[...]
    t = t_ref[...]
    for out_ref, w_ref, b_ref in ((q_ref, wq_ref, bq_ref),
                                  (k_ref, wk_ref, bk_ref),
                                  (v_ref, wv_ref, bv_ref)):
        acc = jnp.zeros(x.shape, jnp.float32)
        for tt in range(T):
            sel = (t == tt).astype(jnp.float32)
            y = jnp.dot(x, w_ref[tt], preferred_element_type=jnp.float32)
            acc = acc + sel * (y + b_ref[tt:tt + 1, :])
        out_ref[...] = acc


def _project(node_inp, node_type2d, Wq, bq, Wk, bk, Wv, bv):
    B = 1000
    grid = (N // B,)
    row = pl.BlockSpec((B, DIM), lambda i: (i, 0))
    tspec = pl.BlockSpec((B, 1), lambda i: (i, 0))
    wspec = pl.BlockSpec((T, DIM, DIM), lambda i: (0, 0, 0))
    bspec = pl.BlockSpec((T, DIM), lambda i: (0, 0))
    out = jax.ShapeDtypeStruct((N, DIM), jnp.float32)
    return pl.pallas_call(
        _proj_body,
        grid=grid,
        in_specs=[row, tspec, wspec, bspec, wspec, bspec, wspec, bspec],
        out_specs=[row, row, row],
        out_shape=[out, out, out],
    )(node_inp, node_type2d, Wq, bq, Wk, bk, Wv, bv)


@functools.partial(
    pl.kernel,
    mesh=_mesh,
    compiler_params=pltpu.CompilerParams(use_tc_tiling_on_sc=False,
                                         needs_layout_passes=False),
    out_type=jax.ShapeDtypeStruct((NC * N, ACC_W), jnp.float32),
    scratch_types=[
        pltpu.VMEM_SHARED((NPAD, ACC_W), jnp.float32),
        pltpu.VMEM((R * 3, DIM), jnp.float32),
        pltpu.VMEM((R * 3, DIM), jnp.float32),
        pltpu.VMEM((R * 3, 16), jnp.float32),
        pltpu.VMEM((C,), jnp.int32),
        pltpu.VMEM((C,), jnp.int32),
        pltpu.VMEM((C,), jnp.int32),
        pltpu.VMEM((C,), jnp.int32),
        pltpu.VMEM((C,), jnp.int32),
        pltpu.VMEM((C, DIM), jnp.float32),
        pltpu.VMEM((C, DIM), jnp.float32),
        pltpu.VMEM((C, DIM), jnp.float32),
        pltpu.VMEM((C, ACC_W), jnp.float32),
        pltpu.SemaphoreType.DMA,
        pltpu.SemaphoreType.DMA,
        pltpu.SemaphoreType.DMA,
        pltpu.SemaphoreType.DMA,
        pltpu.SemaphoreType.DMA,
        pltpu.SemaphoreType.DMA,
        pltpu.SemaphoreType.DMA,
    ],
)
def _edge_kernel(src_hbm, dst_hbm, et_hbm, sg_hbm, q_hbm, k_hbm, v_hbm,
                 cs_hbm, cv_hbm, eb_hbm, out_hbm,
                 acc_sh, cs_v, cv_v, eb_v, src_v, dst_v, et_v, sg_v, ci_v,
                 q_rows, k_rows, v_rows, msg_b,
                 sem0, sem1, sem2, sem3, sem4, sem5, sem6):
    c = lax.axis_index("c")
    s = lax.axis_index("s")
    wid = c * NS + s

    pltpu.async_copy(cs_hbm, cs_v, sem0).wait()
    pltpu.async_copy(cv_hbm, cv_v, sem1).wait()
    pltpu.async_copy(eb_hbm, eb_v, sem2).wait()

    zero16 = jnp.zeros((16,), jnp.float32)

    @pl.loop(0, C)
    def _(i):
        for j in range(ACC_W // 16):
            msg_b[i, pl.ds(j * 16, 16)] = zero16

    @pl.loop(0, ROWS_PT - C, step=C)
    def _(i):
        pltpu.sync_copy(msg_b, acc_sh.at[pl.ds(s * ROWS_PT + i, C)])

    _rem = ROWS_PT % C if ROWS_PT % C else C
    pltpu.sync_copy(msg_b.at[pl.ds(0, _rem)],
                    acc_sh.at[pl.ds(s * ROWS_PT + ROWS_PT - _rem, _rem)])

    plsc.subcore_barrier()

    base0 = wid * EPW
    iot = lax.iota(jnp.int32, 16)

    @pl.loop(0, NCHUNK)
    def _(j):
        base = base0 + j * C
        h_src = pltpu.async_copy(src_hbm.at[pl.ds(base, C)], src_v, sem0)
        h_dst = pltpu.async_copy(dst_hbm.at[pl.ds(base, C)], dst_v, sem1)
        h_et = pltpu.async_copy(et_hbm.at[pl.ds(base, C)], et_v, sem2)
        h_sg = pltpu.async_copy(sg_hbm.at[pl.ds(base, C)], sg_v, sem3)
        h_src.wait()
        h_dst.wait()
        g_q = pltpu.async_copy(q_hbm.at[dst_v], q_rows, sem4)
        g_k = pltpu.async_copy(k_hbm.at[src_v], k_rows, sem5)
        g_v = pltpu.async_copy(v_hbm.at[src_v], v_rows, sem6)
        h_et.wait()
        h_sg.wait()

        @pl.loop(0, C, step=16)
        def _(i):
            et = et_v[pl.ds(i, 16)]
            sg = sg_v[pl.ds(i, 16)]
            sidx = jnp.where(sg == -1, 0, jnp.where(sg == 1, 1, 2))
            ci_v[pl.ds(i, 16)] = et * 3 + sidx

        g_q.wait()
        g_k.wait()
        g_v.wait()

        @pl.loop(0, C, step=16)
        def _(eb):
            civ = ci_v[pl.ds(eb, 16)]
            for kk in range(16):
                e = eb + kk
                ci = civ[kk]
                den = zero16
                for hh in range(H):
                    sl = pl.ds(hh * 16, 16)
                    prod = q_rows[e, sl] * k_rows[e, sl] * cs_v[ci, sl]
                    sc = jnp.sum(prod)
                    exv = jnp.exp(jnp.broadcast_to(sc, (16,)))
                    msg_b[e, sl] = v_rows[e, sl] * cv_v[ci, sl] * exv
                    den = jnp.where(iot == hh, exv, den)
                msg_b[e, pl.ds(DIM, 16)] = den * eb_v[ci, pl.ds(0, 16)]

        pltpu.sync_copy(msg_b, acc_sh.at[dst_v], add=True)

    plsc.subcore_barrier()
    pltpu.sync_copy(acc_sh.at[pl.ds(s * ROWS_PT, ROWS_PT)],
                    out_hbm.at[pl.ds(c * N + s * ROWS_PT, ROWS_PT)])


def _final_body(a0_ref, a1_ref, x_ref, t_ref, sel_ref, alpha_ref,
                gamma_ref, beta_ref, o_ref):
    num = a0_ref[:, :DIM] + a1_ref[:, :DIM]
    den8 = a0_ref[:, DIM:DIM + H] + a1_ref[:, DIM:DIM + H]
    den = jnp.dot(den8, sel_ref[...], preferred_element_type=jnp.float32)
    out = num / jnp.maximum(den, 1e-16)
    x = x_ref[...]
    t = t_ref[...]
    iota_t = lax.broadcasted_iota(jnp.int32, (t.shape[0], T), 1)
    onehot = (t == iota_t).astype(jnp.float32)
    arow = jnp.dot(onehot, alpha_ref[...], preferred_element_type=jnp.float32)
    grow = jnp.dot(onehot, gamma_ref[...], preferred_element_type=jnp.float32)
    brow = jnp.dot(onehot, beta_ref[...], preferred_element_type=jnp.float32)
    hm = arow * out + (1.0 - arow) * x
    mu = jnp.mean(hm, axis=1, keepdims=True)
    var = jnp.mean((hm - mu) ** 2, axis=1, keepdims=True)
    o_ref[...] = (hm - mu) * lax.rsqrt(var + 1e-5) * grow + brow


def _finalize(acc0, acc1, node_inp, node_type2d, sel8, alpha_mat, gamma, beta):
    B = 1000
    grid = (N // B,)
    aspec = pl.BlockSpec((B, ACC_W), lambda i: (i, 0))
    row = pl.BlockSpec((B, DIM), lambda i: (i, 0))
    tspec = pl.BlockSpec((B, 1), lambda i: (i, 0))
    sspec = pl.BlockSpec((H, DIM), lambda i: (0, 0))
    pspec = pl.BlockSpec((T, DIM), lambda i: (0, 0))
    return pl.pallas_call(
        _final_body,
        grid=grid,
        in_specs=[aspec, aspec, row, tspec, sspec, pspec, pspec, pspec],
        out_specs=row,
        out_shape=jax.ShapeDtypeStruct((N, DIM), jnp.float32),
    )(acc0, acc1, node_inp, node_type2d, sel8, alpha_mat, gamma, beta)


def kernel(node_inp, node_type, edge_index, edge_type, edge_sign,
           Wq, bq, Wk, bk, Wv, bv, rel_q, rel_k, rel_v,
           sign_k_fixed, sign_v_fixed, sign_k_neutral, sign_v_neutral,
           rel_bias, skip, gamma, beta):
    src = edge_index[0].astype(jnp.int32)
    dst = edge_index[1].astype(jnp.int32)
    et = edge_type.astype(jnp.int32)
    sg = edge_sign.astype(jnp.int32)
    pad = EP - E
    src = jnp.concatenate([src, jnp.zeros((pad,), jnp.int32)])
    dst = jnp.concatenate([dst, jnp.full((pad,), N, jnp.int32)])
    et = jnp.concatenate([et, jnp.zeros((pad,), jnp.int32)])
    sg = jnp.concatenate([sg, jnp.zeros((pad,), jnp.int32)])
    node_type2d = node_type.astype(jnp.int32).reshape(N, 1)

    sk_all = jnp.concatenate([sign_k_fixed, sign_k_neutral[None]], axis=0)
    sv_all = jnp.concatenate([sign_v_fixed, sign_v_neutral[None]], axis=0)
    eb = jnp.exp(rel_bias)
    cs24 = ((rel_q * rel_k)[:, None] * sk_all[None]
            / math.sqrt(DK)).reshape(R * 3, DIM)
    cv24 = (rel_v[:, None] * sv_all[None]
            * eb[:, None, :, None]).reshape(R * 3, DIM)
    eb24 = jnp.concatenate(
        [jnp.tile(eb[:, None], (1, 3, 1)).reshape(R * 3, H),
         jnp.zeros((R * 3, 8), jnp.float32)], axis=1)

    alphas = jax.nn.sigmoid(skip)
    alpha_mat = jnp.broadcast_to(alphas[:, None], (T, DIM)).astype(jnp.float32)
    sel8 = jnp.kron(jnp.eye(H, dtype=jnp.float32),
                    jnp.ones((1, DK), jnp.float32))

    q, k, v = _project(node_inp, node_type2d, Wq, bq, Wk, bk, Wv, bv)
    zrows = jnp.zeros((NPAD - N, DIM), jnp.float32)
    q = jnp.concatenate([q, zrows])
    k = jnp.concatenate([k, zrows])
    v = jnp.concatenate([v, zrows])
    acc = _edge_kernel(src, dst, et, sg, q, k, v, cs24, cv24, eb24)
    return _finalize(acc[:N], acc[N:], node_inp, node_type2d,
                     sel8, alpha_mat, gamma, beta)
